# Initial kernel scaffold; baseline (speedup 1.0000x reference)
import jax
import jax.numpy as jnp
from jax.experimental import pallas as pl


def kernel(x, edge_index, batch, W1, b1, W2, b2, W3, b3, W4, b4, W5, b5, beta1, beta2, Wl, bl):
    raise NotImplementedError("write your pallas kernel here")


# speedup vs baseline: 10.8742x; 10.8742x over previous
import functools

import jax
import jax.numpy as jnp
from jax import lax
from jax.experimental import pallas as pl
from jax.experimental.pallas import tpu as pltpu
from jax.experimental.pallas import tpu_sc as plsc

_NC, _NS = 2, 16
_NW = _NC * _NS
_C = 128
_G = 64

_f32 = jnp.float32


def _win(nchunks):
    maxcnt = (nchunks + _NW - 1) // _NW
    return ((maxcnt + 7 + 7) // 8) * 8


def _rows_pad(nchunks):
    astart_max = ((((_NW - 1) * nchunks) // _NW) // 8) * 8
    return max(((nchunks + 7) // 8) * 8, astart_max + _win(nchunks))


def _worker_chunks(nchunks):
    cid = lax.axis_index("c")
    sid = lax.axis_index("s")
    w = sid * _NC + cid
    start = (w * nchunks) // _NW
    cnt = ((w + 1) * nchunks) // _NW - start
    astart = (start // 8) * 8
    return cid, sid, start, astart, cnt


def _mesh():
    return plsc.VectorSubcoreMesh(core_axis_name="c", subcore_axis_name="s")


_SC_PARAMS = dict(
    compiler_params=pltpu.CompilerParams(use_tc_tiling_on_sc=False),
)


def _sc_deg(dst2, ones_ch, zeros_nh, nch):
    n, h = zeros_nh.shape
    win = _win(nch)

    @functools.partial(
        pl.kernel,
        out_type=jax.ShapeDtypeStruct((_NC, n, h), _f32),
        mesh=_mesh(),
        scratch_types=[
            pltpu.VMEM((win, _C), jnp.int32),
            pltpu.VMEM((_C, h), _f32),
            pltpu.VMEM_SHARED((n, h), _f32),
        ],
        **_SC_PARAMS,
    )
    def k(dst_h, ones_h, zeros_h, out_h, idxv, onesv, degsp):
        cid, sid, start, astart, cnt = _worker_chunks(nch)
        off0 = start - astart
        pltpu.sync_copy(dst_h.at[pl.ds(astart, win)], idxv)
        pltpu.sync_copy(ones_h, onesv)

        @pl.when(sid == 0)
        def _():
            pltpu.sync_copy(zeros_h, degsp)

        plsc.subcore_barrier()

        def body(j, c):
            pltpu.sync_copy(onesv, degsp.at[idxv.at[off0 + j]], add=True)
            return c

        lax.fori_loop(0, cnt, body, 0)
        plsc.subcore_barrier()

        @pl.when(sid == 0)
        def _():
            pltpu.sync_copy(degsp, out_h.at[cid])

    return k(dst2, ones_ch, zeros_nh)


def _sc_gcn(g, src2, dst2, zeros_nh, nch):
    n, h = g.shape
    win = _win(nch)

    @functools.partial(
        pl.kernel,
        out_type=jax.ShapeDtypeStruct((_NC, n, h), _f32),
        mesh=_mesh(),
        scratch_types=[
            pltpu.VMEM((win, _C), jnp.int32),
            pltpu.VMEM((win, _C), jnp.int32),
            pltpu.VMEM((_C, h), _f32),
            pltpu.VMEM_SHARED((n, h), _f32),
        ],
        **_SC_PARAMS,
    )
    def k(g_h, src_h, dst_h, zeros_h, out_h, sidx, didx, rows, accsp):
        cid, sid, start, astart, cnt = _worker_chunks(nch)
        off0 = start - astart
        pltpu.sync_copy(src_h.at[pl.ds(astart, win)], sidx)
        pltpu.sync_copy(dst_h.at[pl.ds(astart, win)], didx)

        @pl.when(sid == 0)
        def _():
            pltpu.sync_copy(zeros_h, accsp)

        plsc.subcore_barrier()

        def body(j, c):
            pltpu.sync_copy(g_h.at[sidx.at[off0 + j]], rows)
            pltpu.sync_copy(rows, accsp.at[didx.at[off0 + j]], add=True)
            return c

        lax.fori_loop(0, cnt, body, 0)
        plsc.subcore_barrier()

        @pl.when(sid == 0)
        def _():
            pltpu.sync_copy(accsp, out_h.at[cid])

    return k(g, src2, dst2, zeros_nh)


def _sc_ag_gather(xn, x, src2, dst2, nch):
    n, h = xn.shape
    e = nch * _C
    win = _win(nch)

    @functools.partial(
        pl.kernel,
        out_type=[
            jax.ShapeDtypeStruct((e, h), _f32),
            jax.ShapeDtypeStruct((e, h), _f32),
            jax.ShapeDtypeStruct((e, h), _f32),
        ],
        mesh=_mesh(),
        scratch_types=[
            pltpu.VMEM((win, _C), jnp.int32),
            pltpu.VMEM((win, _C), jnp.int32),
            pltpu.VMEM((_C, h), _f32),
        ],
        **_SC_PARAMS,
    )
    def k(xn_h, x_h, src_h, dst_h, xns_h, xnd_h, xs_h, sidx, didx, rows):
        cid, sid, start, astart, cnt = _worker_chunks(nch)
        off0 = start - astart
        pltpu.sync_copy(src_h.at[pl.ds(astart, win)], sidx)
        pltpu.sync_copy(dst_h.at[pl.ds(astart, win)], didx)

        def body(j, c):
            row0 = (start + j) * _C
            pltpu.sync_copy(xn_h.at[sidx.at[off0 + j]], rows)
            pltpu.sync_copy(rows, xns_h.at[pl.ds(row0, _C)])
            pltpu.sync_copy(xn_h.at[didx.at[off0 + j]], rows)
            pltpu.sync_copy(rows, xnd_h.at[pl.ds(row0, _C)])
            pltpu.sync_copy(x_h.at[sidx.at[off0 + j]], rows)
            pltpu.sync_copy(rows, xs_h.at[pl.ds(row0, _C)])
            return c

        lax.fori_loop(0, cnt, body, 0)

    return k(xn, x, src2, dst2)


def _sc_ag_scatter(msgaug, dst2, zeros_n2h, nch):
    n, h2 = zeros_n2h.shape
    win = _win(nch)

    @functools.partial(
        pl.kernel,
        out_type=jax.ShapeDtypeStruct((_NC, n, h2), _f32),
        mesh=_mesh(),
        scratch_types=[
            pltpu.VMEM((win, _C), jnp.int32),
            pltpu.VMEM((_C, h2), _f32),
            pltpu.VMEM_SHARED((n, h2), _f32),
        ],
        **_SC_PARAMS,
    )
    def k(msg_h, dst_h, zeros_h, out_h, didx, rows, numsp):
        cid, sid, start, astart, cnt = _worker_chunks(nch)
        off0 = start - astart
        pltpu.sync_copy(dst_h.at[pl.ds(astart, win)], didx)

        @pl.when(sid == 0)
        def _():
            pltpu.sync_copy(zeros_h, numsp)

        plsc.subcore_barrier()

        def body(j, c):
            row0 = (start + j) * _C
            pltpu.sync_copy(msg_h.at[pl.ds(row0, _C)], rows)
            pltpu.sync_copy(rows, numsp.at[didx.at[off0 + j]], add=True)
            return c

        lax.fori_loop(0, cnt, body, 0)
        plsc.subcore_barrier()

        @pl.when(sid == 0)
        def _():
            pltpu.sync_copy(numsp, out_h.at[cid])

    return k(msgaug, dst2, zeros_n2h)


def _tc_pre(x, W1, degp1):
    n = x.shape[0]
    h = W1.shape[1]

    def body(x_ref, w_ref, degp_ref, g_ref, dinv_ref):
        dp = degp_ref[...]
        deg = dp[0] + dp[1] + 1.0
        dinv = 1.0 / jnp.sqrt(deg)
        dinv_ref[...] = dinv
        g_ref[...] = jnp.dot(x_ref[...], w_ref[...],
                             preferred_element_type=_f32) * dinv

    return pl.pallas_call(
        body,
        out_shape=[jax.ShapeDtypeStruct((n, h), _f32),
                   jax.ShapeDtypeStruct((n, 1), _f32)],
    )(x, W1, degp1)


def _tc_gcn_post(accp, g, dinv, b, Wn):
    n, h = g.shape

    def body(accp_ref, g_ref, dinv_ref, b_ref, wn_ref, out_ref):
        ap = accp_ref[...]
        dinv = dinv_ref[...]
        hh = jnp.tanh((ap[0] + ap[1] + g_ref[...]) * dinv + b_ref[...])
        out_ref[...] = jnp.dot(hh, wn_ref[...],
                               preferred_element_type=_f32) * dinv

    return pl.pallas_call(
        body, out_shape=jax.ShapeDtypeStruct((n, h), _f32),
    )(accp, g, dinv, b, Wn)


def _tc_gcn_final(accp, g, dinv, b):
    n, h = g.shape

    def body(accp_ref, g_ref, dinv_ref, b_ref, xn_ref, x_ref):
        ap = accp_ref[...]
        hh = jnp.tanh((ap[0] + ap[1] + g_ref[...]) * dinv_ref[...] + b_ref[...])
        r = jnp.sqrt(jnp.sum(hh * hh, axis=1, keepdims=True))
        xn_ref[...] = hh / jnp.maximum(r, 1e-12)
        x_ref[...] = hh

    return pl.pallas_call(
        body,
        out_shape=[jax.ShapeDtypeStruct((n, h), _f32),
                   jax.ShapeDtypeStruct((n, h), _f32)],
    )(accp, g, dinv, b)


def _tc_edge(xns, xnd, xs, beta):
    e, h = xns.shape
    be = 8000
    grid = e // be

    def body(xns_ref, xnd_ref, xs_ref, beta_ref, msg_ref):
        a = jnp.sum(xns_ref[...] * xnd_ref[...], axis=1, keepdims=True)
        w = jnp.exp(a * beta_ref[...])
        msg_ref[...] = jnp.concatenate(
            [xs_ref[...] * w, w, jnp.zeros((be, h - 1), _f32)], axis=1)

    rowspec = pl.BlockSpec((be, h), lambda i: (i, 0))
    return pl.pallas_call(
        body,
        grid=(grid,),
        in_specs=[rowspec, rowspec, rowspec,
                  pl.BlockSpec((1, 1), lambda i: (0, 0))],
        out_specs=pl.BlockSpec((be, 2 * h), lambda i: (i, 0)),
        out_shape=jax.ShapeDtypeStruct((e, 2 * h), _f32),
    )(xns, xnd, xs, beta)


def _agnn_finish(ndp_ref, xn_ref, x_ref, beta_ref, h):
    xnv = xn_ref[...]
    selfw = jnp.exp(jnp.sum(xnv * xnv, axis=1, keepdims=True) * beta_ref[...])
    nd = ndp_ref[...]
    nd = nd[0] + nd[1]
    num = nd[:, :h] + selfw * x_ref[...]
    den = nd[:, h:h + 1] + selfw + 1e-16
    return jnp.tanh(num / den)


def _tc_ag_post(ndp, xn, x, beta):
    n, h = xn.shape

    def body(ndp_ref, xn_ref, x_ref, beta_ref, xn2_ref, x2_ref):
        hh = _agnn_finish(ndp_ref, xn_ref, x_ref, beta_ref, h)
        r2 = jnp.sqrt(jnp.sum(hh * hh, axis=1, keepdims=True))
        xn2_ref[...] = hh / jnp.maximum(r2, 1e-12)
        x2_ref[...] = hh

    return pl.pallas_call(
        body,
        out_shape=[jax.ShapeDtypeStruct((n, h), _f32),
                   jax.ShapeDtypeStruct((n, h), _f32)],
    )(ndp, xn, x, beta)


def _tc_final(ndp, xn, x, beta, batch2, Wl, bl):
    n, h = xn.shape

    def body(ndp_ref, xn_ref, x_ref, beta_ref, batch_ref, wl_ref, bl_ref,
             out_ref):
        hh = _agnn_finish(ndp_ref, xn_ref, x_ref, beta_ref, h)
        gi = lax.broadcasted_iota(jnp.int32, (n, _G), 1)
        mask = (batch_ref[...] == gi).astype(_f32)
        sums = lax.dot_general(mask, hh, (((0,), (0,)), ((), ())),
                               preferred_element_type=_f32)
        cnt = lax.dot_general(mask, jnp.ones((n, 1), _f32),
                              (((0,), (0,)), ((), ())),
                              preferred_element_type=_f32)
        pooled = sums / jnp.maximum(cnt, 1.0)
        mu = jnp.mean(pooled, axis=1, keepdims=True)
        var = jnp.mean((pooled - mu) ** 2, axis=1, keepdims=True)
        normed = (pooled - mu) / jnp.sqrt(var + 1e-5)
        out_ref[...] = jnp.dot(normed, wl_ref[...],
                               preferred_element_type=_f32) + bl_ref[...]

    return pl.pallas_call(
        body, out_shape=jax.ShapeDtypeStruct((_G, 1), _f32),
    )(ndp, xn, x, beta, batch2, Wl, bl)


def kernel(x, edge_index, batch, W1, b1, W2, b2, W3, b3, W4, b4, W5, b5,
           beta1, beta2, Wl, bl):
    n, _ = x.shape
    h = W1.shape[1]
    e = edge_index.shape[1]
    nch = e // _C

    pad_rows = _rows_pad(nch) - nch
    src2 = jnp.pad(edge_index[0].reshape(nch, _C), ((0, pad_rows), (0, 0)))
    dst2 = jnp.pad(edge_index[1].reshape(nch, _C), ((0, pad_rows), (0, 0)))
    batch2 = batch.reshape(n, 1)
    zeros_nh = jnp.zeros((n, h), _f32)
    zeros_n2h = jnp.zeros((n, 2 * h), _f32)
    ones_ch = jnp.ones((_C, h), _f32)

    degp = _sc_deg(dst2, ones_ch, zeros_nh, nch)
    g, dinv = _tc_pre(x, W1, degp[:, :, 0:1])

    for Wn, b in ((W2, b1), (W3, b2), (W4, b3), (W5, b4)):
        accp = _sc_gcn(g, src2, dst2, zeros_nh, nch)
        g = _tc_gcn_post(accp, g, dinv, b.reshape(1, h), Wn)

    accp = _sc_gcn(g, src2, dst2, zeros_nh, nch)
    xn, xv = _tc_gcn_final(accp, g, dinv, b5.reshape(1, h))

    xns, xnd, xs = _sc_ag_gather(xn, xv, src2, dst2, nch)
    msgaug = _tc_edge(xns, xnd, xs, beta1.reshape(1, 1))
    ndp = _sc_ag_scatter(msgaug, dst2, zeros_n2h, nch)
    xn, xv = _tc_ag_post(ndp, xn, xv, beta1.reshape(1, 1))

    xns, xnd, xs = _sc_ag_gather(xn, xv, src2, dst2, nch)
    msgaug = _tc_edge(xns, xnd, xs, beta2.reshape(1, 1))
    ndp = _sc_ag_scatter(msgaug, dst2, zeros_n2h, nch)

    return _tc_final(ndp, xn, xv, beta2.reshape(1, 1), batch2, Wl, bl)
